# baseline (device time: 39156 ns/iter reference)
import jax
import jax.numpy as jnp
from jax import lax
from jax.experimental import pallas as pl
from jax.experimental.pallas import tpu as pltpu


def kernel(Q, K, V):
    b, sq, h, d = Q.shape
    _, skv, _, _ = K.shape
    scale = d ** -0.5
    comm_w = 128

    def body(q_ref, k_ref, v_ref, o_ref, comm_ref, send_sem, recv_sem):
        my_x = lax.axis_index("x")
        my_y = lax.axis_index("y")
        my_z = lax.axis_index("z")
        peer = (1 - my_x, my_y, my_z)

        barrier_sem = pltpu.get_barrier_semaphore()
        pl.semaphore_signal(
            barrier_sem, inc=1, device_id=peer,
            device_id_type=pl.DeviceIdType.MESH,
        )
        pl.semaphore_wait(barrier_sem, 1)

        q = q_ref[:, 0, :, :]
        k = k_ref[...]
        v = v_ref[...]

        s = jnp.sum(k * q[:, None, :, :], axis=-1) * scale
        m = jnp.max(s, axis=1)
        p = jnp.exp(s - m[:, None, :])
        l = jnp.sum(p, axis=1)
        o = jnp.sum(p[:, :, :, None] * v, axis=1)

        comm_ref[0, :, :, 0:d] = o
        comm_ref[0, :, :, d : d + 1] = m[:, :, None]
        comm_ref[0, :, :, d + 1 : d + 2] = l[:, :, None]

        rdma = pltpu.make_async_remote_copy(
            src_ref=comm_ref.at[0],
            dst_ref=comm_ref.at[1],
            send_sem=send_sem,
            recv_sem=recv_sem,
            device_id=peer,
            device_id_type=pl.DeviceIdType.MESH,
        )
        rdma.start()
        rdma.wait()

        o2 = comm_ref[1, :, :, 0:d]
        m2 = comm_ref[1, :, :, d : d + 1][:, :, 0]
        l2 = comm_ref[1, :, :, d + 1 : d + 2][:, :, 0]

        mg = jnp.maximum(m, m2)
        ca = jnp.exp(m - mg)
        cb = jnp.exp(m2 - mg)
        lg = l * ca + l2 * cb
        og = (o * ca[:, :, None] + o2 * cb[:, :, None]) / lg[:, :, None]
        o_ref[:, 0, :, :] = og

    return pl.pallas_call(
        body,
        out_shape=jax.ShapeDtypeStruct((b, sq, h, d), jnp.float32),
        in_specs=[
            pl.BlockSpec(memory_space=pltpu.VMEM),
            pl.BlockSpec(memory_space=pltpu.VMEM),
            pl.BlockSpec(memory_space=pltpu.VMEM),
        ],
        out_specs=pl.BlockSpec(memory_space=pltpu.VMEM),
        scratch_shapes=[
            pltpu.VMEM((2, b, h, comm_w), jnp.float32),
            pltpu.SemaphoreType.DMA,
            pltpu.SemaphoreType.DMA,
        ],
        compiler_params=pltpu.CompilerParams(collective_id=0),
    )(Q, K, V)
